# Initial kernel scaffold; baseline (speedup 1.0000x reference)
import jax
import jax.numpy as jnp
from jax.experimental import pallas as pl


def kernel(x, edge_index, batch, W1, b1, W2, b2):
    raise NotImplementedError("write your pallas kernel here")


# speedup vs baseline: 33.3779x; 33.3779x over previous
import functools

import jax
import jax.numpy as jnp
from jax import lax
from jax.experimental import pallas as pl
from jax.experimental.pallas import tpu as pltpu
from jax.experimental.pallas import tpu_sc as plsc

N = 10000
E = 320000
F = 128
HP = 8
G = 8

NC = 2
NS = 16
NW = NC * NS
EPW = E // NW
K = 80
NCH = EPW // K
RPT = N // NS

_mesh = plsc.VectorSubcoreMesh(core_axis_name="c", subcore_axis_name="s")


def _make_sc_agg(with_gather: bool):
    def body(edges_hbm, table_hbm, zeros_hbm, out_hbm,
             row_all, col_all, msg2, acc, sem_g):
        cid = lax.axis_index("c")
        sid = lax.axis_index("s")
        wid = cid * NS + sid

        pltpu.sync_copy(zeros_hbm.at[pl.ds(sid * RPT, RPT)],
                        acc.at[pl.ds(sid * RPT, RPT)])

        pltpu.sync_copy(edges_hbm.at[0, wid], row_all)
        pltpu.sync_copy(edges_hbm.at[1, wid], col_all)

        if not with_gather:
            pltpu.sync_copy(table_hbm.at[pl.ds(0, K)], msg2.at[0])

        plsc.subcore_barrier()

        if with_gather:
            pltpu.async_copy(table_hbm.at[row_all.at[0]], msg2.at[0],
                             sem_g).wait()

            def step(j, _):
                cur = lax.rem(j, 2)
                nxt = lax.rem(j + 1, 2)

                @pl.when(j + 1 < NCH)
                def _():
                    pltpu.async_copy(table_hbm.at[row_all.at[j + 1]],
                                     msg2.at[nxt], sem_g)

                pltpu.sync_copy(msg2.at[cur], acc.at[col_all.at[j]], add=True)

                @pl.when(j + 1 < NCH)
                def _():
                    pltpu.make_async_copy(table_hbm.at[row_all.at[j + 1]],
                                          msg2.at[nxt], sem_g).wait()
                return 0

            lax.fori_loop(0, NCH, step, 0)
        else:
            def step(j, _):
                pltpu.sync_copy(msg2.at[0], acc.at[col_all.at[j]], add=True)
                return 0

            lax.fori_loop(0, NCH, step, 0)

        plsc.subcore_barrier()

        pltpu.sync_copy(acc.at[pl.ds(sid * RPT, RPT)],
                        out_hbm.at[cid, pl.ds(sid * RPT, RPT)])

    return pl.kernel(
        body,
        out_type=jax.ShapeDtypeStruct((NC, N, HP), jnp.float32),
        mesh=_mesh,
        scratch_types=[
            pltpu.VMEM((NCH, K), jnp.int32),
            pltpu.VMEM((NCH, K), jnp.int32),
            pltpu.VMEM((2, K, HP), jnp.float32),
            pltpu.VMEM_SHARED((N, HP), jnp.float32),
            pltpu.SemaphoreType.DMA,
        ],
        compiler_params=pltpu.CompilerParams(use_tc_tiling_on_sc=False),
    )


_sc_agg = _make_sc_agg(True)
_sc_deg = _make_sc_agg(False)


_RB = 1000


def _tc_b_body(x_ref, w_ref, degp_ref, y_ref, dinv_ref):
    xw = jnp.dot(x_ref[...], w_ref[...], preferred_element_type=jnp.float32)
    deg = degp_ref[0, :, 0] + degp_ref[1, :, 0] + 1.0
    dinv = lax.rsqrt(deg)[:, None]
    dinv8 = jnp.broadcast_to(dinv, (x_ref.shape[0], HP))
    dinv_ref[...] = dinv8
    y_ref[...] = xw * dinv8


def _tc_b(x, w1p, degp):
    return pl.pallas_call(
        _tc_b_body,
        grid=(N // _RB,),
        in_specs=[
            pl.BlockSpec((_RB, F), lambda i: (i, 0)),
            pl.BlockSpec((F, HP), lambda i: (0, 0)),
            pl.BlockSpec((NC, _RB, HP), lambda i: (0, i, 0)),
        ],
        out_specs=[
            pl.BlockSpec((_RB, HP), lambda i: (i, 0)),
            pl.BlockSpec((_RB, HP), lambda i: (i, 0)),
        ],
        out_shape=[
            jax.ShapeDtypeStruct((N, HP), jnp.float32),
            jax.ShapeDtypeStruct((N, HP), jnp.float32),
        ],
    )(x, w1p, degp)


def _tc_d_body(a_ref, y_ref, dinv_ref, b1_ref, z_ref):
    dinv8 = dinv_ref[...]
    agg = a_ref[0] + a_ref[1] + y_ref[...]
    h = jnp.maximum(dinv8 * agg + b1_ref[...], 0.0)
    z_ref[...] = dinv8 * h


def _tc_d(acc1, y, dinv8, b1p):
    return pl.pallas_call(
        _tc_d_body,
        grid=(N // _RB,),
        in_specs=[
            pl.BlockSpec((NC, _RB, HP), lambda i: (0, i, 0)),
            pl.BlockSpec((_RB, HP), lambda i: (i, 0)),
            pl.BlockSpec((_RB, HP), lambda i: (i, 0)),
            pl.BlockSpec((1, HP), lambda i: (0, 0)),
        ],
        out_specs=pl.BlockSpec((_RB, HP), lambda i: (i, 0)),
        out_shape=jax.ShapeDtypeStruct((N, HP), jnp.float32),
    )(acc1, y, dinv8, b1p)


def _tc_f_body(a_ref, z_ref, dinv_ref, batch_ref, w2_ref, b2_ref, out_ref):
    q = dinv_ref[...] * (a_ref[0] + a_ref[1] + z_ref[...])
    gids = lax.broadcasted_iota(jnp.int32, (G, N), 0)
    onehot = (batch_ref[...] == gids).astype(jnp.float32)
    s = jnp.dot(onehot, q, preferred_element_type=jnp.float32)
    cnt = jnp.sum(onehot, axis=1, keepdims=True)
    pooled = s / jnp.maximum(cnt, 1.0)
    logits = jnp.dot(pooled[:, :6], w2_ref[...],
                     preferred_element_type=jnp.float32) + b2_ref[...]
    logits = jnp.where(cnt > 0.0, logits, 0.0)
    m = jnp.max(logits, axis=1, keepdims=True)
    e = logits - m
    out_ref[...] = e - jnp.log(jnp.sum(jnp.exp(e), axis=1, keepdims=True))


def _tc_f(acc2, z, dinv8, batch2d, w2, b2r):
    return pl.pallas_call(
        _tc_f_body,
        out_shape=jax.ShapeDtypeStruct((G, 10), jnp.float32),
    )(acc2, z, dinv8, batch2d, w2, b2r)


def kernel(x, edge_index, batch, W1, b1, W2, b2):
    edges_r = edge_index.reshape(2, NW, NCH, K)
    zeros8 = jnp.zeros((N, HP), jnp.float32)
    ones_k = jnp.ones((K, HP), jnp.float32)

    w1p = jnp.zeros((F, HP), jnp.float32).at[:, :6].set(W1)
    b1p = jnp.zeros((1, HP), jnp.float32).at[0, :6].set(b1)
    batch2d = batch.reshape(1, N)
    b2r = b2.reshape(1, 10)

    degp = _sc_deg(edges_r, ones_k, zeros8)
    y, dinv8 = _tc_b(x, w1p, degp)
    acc1 = _sc_agg(edges_r, y, zeros8)
    z = _tc_d(acc1, y, dinv8, b1p)
    acc2 = _sc_agg(edges_r, z, zeros8)
    return _tc_f(acc2, z, dinv8, batch2d, W2, b2r)


# speedup vs baseline: 44.7336x; 1.3402x over previous
import jax
import jax.numpy as jnp
from jax import lax
from jax.experimental import pallas as pl
from jax.experimental.pallas import tpu as pltpu
from jax.experimental.pallas import tpu_sc as plsc

N = 10000
E = 320000
F = 128
HP = 8
G = 8

NC = 2
NS = 16
NW = NC * NS
EPW = E // NW
K = 128
NCH = 79
EPWP = NCH * K
NP = N + 16
RPT = NP // NS
NBUF = 6
AHEAD = 3

_mesh = plsc.VectorSubcoreMesh(core_axis_name="c", subcore_axis_name="s")


def _make_sc_agg(with_gather: bool):
    def body(edges_hbm, table_hbm, zeros_hbm, out_hbm,
             row_all, col_all, msgs, acc, sem_g, sem_s):
        cid = lax.axis_index("c")
        sid = lax.axis_index("s")
        wid = cid * NS + sid

        pltpu.sync_copy(zeros_hbm.at[pl.ds(sid * RPT, RPT)],
                        acc.at[pl.ds(sid * RPT, RPT)])
        pltpu.sync_copy(edges_hbm.at[0, wid], row_all)
        pltpu.sync_copy(edges_hbm.at[1, wid], col_all)
        if not with_gather:
            pltpu.sync_copy(table_hbm.at[pl.ds(0, K)], msgs.at[0])
        plsc.subcore_barrier()

        if with_gather:
            for p in range(AHEAD):
                pltpu.async_copy(table_hbm.at[row_all.at[p]], msgs.at[p],
                                 sem_g)

            def step(j, _):
                buf = lax.rem(j, NBUF)
                pltpu.make_async_copy(table_hbm.at[row_all.at[j]],
                                      msgs.at[buf], sem_g).wait()
                pltpu.async_copy(msgs.at[buf], acc.at[col_all.at[j]],
                                 sem_s, add=True)

                @pl.when(j >= AHEAD)
                def _():
                    old = lax.rem(j - AHEAD, NBUF)
                    pltpu.make_async_copy(msgs.at[old],
                                          acc.at[col_all.at[j - AHEAD]],
                                          sem_s).wait()

                @pl.when(j + AHEAD < NCH)
                def _():
                    nbuf = lax.rem(j + AHEAD, NBUF)
                    pltpu.async_copy(table_hbm.at[row_all.at[j + AHEAD]],
                                     msgs.at[nbuf], sem_g)
                return 0

            lax.fori_loop(0, NCH, step, 0)
            for t in range(NCH - AHEAD, NCH):
                pltpu.make_async_copy(msgs.at[t % NBUF],
                                      acc.at[col_all.at[t]], sem_s).wait()
        else:
            def step(j, _):
                pltpu.async_copy(msgs.at[0], acc.at[col_all.at[j]],
                                 sem_s, add=True)

                @pl.when(j >= AHEAD)
                def _():
                    pltpu.make_async_copy(msgs.at[0],
                                          acc.at[col_all.at[j - AHEAD]],
                                          sem_s).wait()
                return 0

            lax.fori_loop(0, NCH, step, 0)
            for t in range(NCH - AHEAD, NCH):
                pltpu.make_async_copy(msgs.at[0], acc.at[col_all.at[t]],
                                      sem_s).wait()

        plsc.subcore_barrier()
        pltpu.sync_copy(acc.at[pl.ds(sid * RPT, RPT)],
                        out_hbm.at[cid, pl.ds(sid * RPT, RPT)])

    return pl.kernel(
        body,
        out_type=jax.ShapeDtypeStruct((NC, NP, HP), jnp.float32),
        mesh=_mesh,
        scratch_types=[
            pltpu.VMEM((NCH, K), jnp.int32),
            pltpu.VMEM((NCH, K), jnp.int32),
            pltpu.VMEM((NBUF, K, HP), jnp.float32),
            pltpu.VMEM_SHARED((NP, HP), jnp.float32),
            pltpu.SemaphoreType.DMA,
            pltpu.SemaphoreType.DMA,
        ],
        compiler_params=pltpu.CompilerParams(use_tc_tiling_on_sc=False),
    )


_sc_agg = _make_sc_agg(True)
_sc_deg = _make_sc_agg(False)


_RB = 1000


def _tc_b_body(x_ref, w_ref, degp_ref, y_ref, dinv_ref):
    xw = jnp.dot(x_ref[...], w_ref[...], preferred_element_type=jnp.float32)
    deg = degp_ref[0, :, 0] + degp_ref[1, :, 0] + 1.0
    dinv = lax.rsqrt(deg)[:, None]
    dinv8 = jnp.broadcast_to(dinv, (x_ref.shape[0], HP))
    dinv_ref[...] = dinv8
    y_ref[...] = xw * dinv8


def _tc_b(x, w1p, degp):
    return pl.pallas_call(
        _tc_b_body,
        grid=(N // _RB,),
        in_specs=[
            pl.BlockSpec((_RB, F), lambda i: (i, 0)),
            pl.BlockSpec((F, HP), lambda i: (0, 0)),
            pl.BlockSpec((NC, _RB, HP), lambda i: (0, i, 0)),
        ],
        out_specs=[
            pl.BlockSpec((_RB, HP), lambda i: (i, 0)),
            pl.BlockSpec((_RB, HP), lambda i: (i, 0)),
        ],
        out_shape=[
            jax.ShapeDtypeStruct((N, HP), jnp.float32),
            jax.ShapeDtypeStruct((N, HP), jnp.float32),
        ],
    )(x, w1p, degp)


def _tc_d_body(a_ref, y_ref, dinv_ref, b1_ref, z_ref):
    dinv8 = dinv_ref[...]
    agg = a_ref[0] + a_ref[1] + y_ref[...]
    h = jnp.maximum(dinv8 * agg + b1_ref[...], 0.0)
    z_ref[...] = dinv8 * h


def _tc_d(acc1, y, dinv8, b1p):
    return pl.pallas_call(
        _tc_d_body,
        grid=(N // _RB,),
        in_specs=[
            pl.BlockSpec((NC, _RB, HP), lambda i: (0, i, 0)),
            pl.BlockSpec((_RB, HP), lambda i: (i, 0)),
            pl.BlockSpec((_RB, HP), lambda i: (i, 0)),
            pl.BlockSpec((1, HP), lambda i: (0, 0)),
        ],
        out_specs=pl.BlockSpec((_RB, HP), lambda i: (i, 0)),
        out_shape=jax.ShapeDtypeStruct((N, HP), jnp.float32),
    )(acc1, y, dinv8, b1p)


def _tc_f_body(a_ref, z_ref, dinv_ref, batch_ref, w2_ref, b2_ref, out_ref):
    q = dinv_ref[...] * (a_ref[0] + a_ref[1] + z_ref[...])
    gids = lax.broadcasted_iota(jnp.int32, (G, N), 0)
    onehot = (batch_ref[...] == gids).astype(jnp.float32)
    s = jnp.dot(onehot, q, preferred_element_type=jnp.float32)
    cnt = jnp.sum(onehot, axis=1, keepdims=True)
    pooled = s / jnp.maximum(cnt, 1.0)
    logits = jnp.dot(pooled[:, :6], w2_ref[...],
                     preferred_element_type=jnp.float32) + b2_ref[...]
    logits = jnp.where(cnt > 0.0, logits, 0.0)
    m = jnp.max(logits, axis=1, keepdims=True)
    e = logits - m
    out_ref[...] = e - jnp.log(jnp.sum(jnp.exp(e), axis=1, keepdims=True))


def _tc_f(acc2, z, dinv8, batch2d, w2, b2r):
    return pl.pallas_call(
        _tc_f_body,
        grid=(1,),
        in_specs=[
            pl.BlockSpec((NC, N, HP), lambda i: (0, 0, 0)),
            pl.BlockSpec((N, HP), lambda i: (0, 0)),
            pl.BlockSpec((N, HP), lambda i: (0, 0)),
            pl.BlockSpec((1, N), lambda i: (0, 0)),
            pl.BlockSpec((6, 10), lambda i: (0, 0)),
            pl.BlockSpec((1, 10), lambda i: (0, 0)),
        ],
        out_specs=pl.BlockSpec((G, 10), lambda i: (0, 0)),
        out_shape=jax.ShapeDtypeStruct((G, 10), jnp.float32),
    )(acc2, z, dinv8, batch2d, w2, b2r)


_PAD16 = 16


def kernel(x, edge_index, batch, W1, b1, W2, b2):
    e3 = edge_index.reshape(2, NW, EPW)
    e3 = jnp.pad(e3, ((0, 0), (0, 0), (0, EPWP - EPW)), constant_values=N)
    edges_r = e3.reshape(2, NW, NCH, K)

    zeros8 = jnp.zeros((NP, HP), jnp.float32)
    ones_k = jnp.ones((K, HP), jnp.float32)
    zpad = jnp.zeros((_PAD16, HP), jnp.float32)

    w1p = jnp.zeros((F, HP), jnp.float32).at[:, :6].set(W1)
    b1p = jnp.zeros((1, HP), jnp.float32).at[0, :6].set(b1)
    batch2d = batch.reshape(1, N)
    b2r = b2.reshape(1, 10)

    degp = _sc_deg(edges_r, ones_k, zeros8)
    y, dinv8 = _tc_b(x, w1p, degp)
    yp = jnp.concatenate([y, zpad], axis=0)
    acc1 = _sc_agg(edges_r, yp, zeros8)
    z = _tc_d(acc1, y, dinv8, b1p)
    zp = jnp.concatenate([z, zpad], axis=0)
    acc2 = _sc_agg(edges_r, zp, zeros8)
    return _tc_f(acc2, z, dinv8, batch2d, W2, b2r)


# speedup vs baseline: 72.7250x; 1.6257x over previous
import jax
import jax.numpy as jnp
from jax import lax
from jax.experimental import pallas as pl
from jax.experimental.pallas import tpu as pltpu
from jax.experimental.pallas import tpu_sc as plsc

N = 10000
E = 320000
F = 128
HP = 8
G = 8

NC = 2
NS = 16
NW = NC * NS
EPW = E // NW
K = 128
NCH = 79
EPWP = NCH * K
NP = N + 16
RPT = NP // NS
NBUF = 6
AHEAD = 3

_mesh = plsc.VectorSubcoreMesh(core_axis_name="c", subcore_axis_name="s")


def _make_sc_agg(with_gather: bool):
    def body(edges_hbm, table_hbm, zeros_hbm, out_hbm,
             row_all, col_all, msgs, acc, table_s, sem_g, sem_s):
        cid = lax.axis_index("c")
        sid = lax.axis_index("s")
        wid = cid * NS + sid

        pltpu.sync_copy(zeros_hbm.at[pl.ds(sid * RPT, RPT)],
                        acc.at[pl.ds(sid * RPT, RPT)])
        pltpu.sync_copy(edges_hbm.at[0, wid], row_all)
        pltpu.sync_copy(edges_hbm.at[1, wid], col_all)
        if with_gather:
            pltpu.sync_copy(table_hbm.at[pl.ds(sid * RPT, RPT)],
                            table_s.at[pl.ds(sid * RPT, RPT)])
        else:
            pltpu.sync_copy(table_hbm.at[pl.ds(0, K)], msgs.at[0])
        plsc.subcore_barrier()

        if with_gather:
            for p in range(AHEAD):
                pltpu.async_copy(table_s.at[row_all.at[p]], msgs.at[p],
                                 sem_g)

            def step(j, _):
                buf = lax.rem(j, NBUF)
                pltpu.make_async_copy(table_s.at[row_all.at[j]],
                                      msgs.at[buf], sem_g).wait()
                pltpu.async_copy(msgs.at[buf], acc.at[col_all.at[j]],
                                 sem_s, add=True)

                @pl.when(j >= AHEAD)
                def _():
                    old = lax.rem(j - AHEAD, NBUF)
                    pltpu.make_async_copy(msgs.at[old],
                                          acc.at[col_all.at[j - AHEAD]],
                                          sem_s).wait()

                @pl.when(j + AHEAD < NCH)
                def _():
                    nbuf = lax.rem(j + AHEAD, NBUF)
                    pltpu.async_copy(table_s.at[row_all.at[j + AHEAD]],
                                     msgs.at[nbuf], sem_g)
                return 0

            lax.fori_loop(0, NCH, step, 0)
            for t in range(NCH - AHEAD, NCH):
                pltpu.make_async_copy(msgs.at[t % NBUF],
                                      acc.at[col_all.at[t]], sem_s).wait()
        else:
            def step(j, _):
                pltpu.async_copy(msgs.at[0], acc.at[col_all.at[j]],
                                 sem_s, add=True)

                @pl.when(j >= AHEAD)
                def _():
                    pltpu.make_async_copy(msgs.at[0],
                                          acc.at[col_all.at[j - AHEAD]],
                                          sem_s).wait()
                return 0

            lax.fori_loop(0, NCH, step, 0)
            for t in range(NCH - AHEAD, NCH):
                pltpu.make_async_copy(msgs.at[0], acc.at[col_all.at[t]],
                                      sem_s).wait()

        plsc.subcore_barrier()
        pltpu.sync_copy(acc.at[pl.ds(sid * RPT, RPT)],
                        out_hbm.at[cid, pl.ds(sid * RPT, RPT)])

    return pl.kernel(
        body,
        out_type=jax.ShapeDtypeStruct((NC, NP, HP), jnp.float32),
        mesh=_mesh,
        scratch_types=[
            pltpu.VMEM((NCH, K), jnp.int32),
            pltpu.VMEM((NCH, K), jnp.int32),
            pltpu.VMEM((NBUF, K, HP), jnp.float32),
            pltpu.VMEM_SHARED((NP, HP), jnp.float32),
            pltpu.VMEM_SHARED((NP, HP), jnp.float32),
            pltpu.SemaphoreType.DMA,
            pltpu.SemaphoreType.DMA,
        ],
        compiler_params=pltpu.CompilerParams(use_tc_tiling_on_sc=False),
    )


_sc_agg = _make_sc_agg(True)
_sc_deg = _make_sc_agg(False)


_RB = 1000


def _tc_b_body(x_ref, w_ref, degp_ref, y_ref, dinv_ref):
    xw = jnp.dot(x_ref[...], w_ref[...], preferred_element_type=jnp.float32)
    deg = degp_ref[0, :, 0] + degp_ref[1, :, 0] + 1.0
    dinv = lax.rsqrt(deg)[:, None]
    dinv8 = jnp.broadcast_to(dinv, (x_ref.shape[0], HP))
    dinv_ref[...] = dinv8
    y_ref[...] = xw * dinv8


def _tc_b(x, w1p, degp):
    return pl.pallas_call(
        _tc_b_body,
        grid=(N // _RB,),
        in_specs=[
            pl.BlockSpec((_RB, F), lambda i: (i, 0)),
            pl.BlockSpec((F, HP), lambda i: (0, 0)),
            pl.BlockSpec((NC, _RB, HP), lambda i: (0, i, 0)),
        ],
        out_specs=[
            pl.BlockSpec((_RB, HP), lambda i: (i, 0)),
            pl.BlockSpec((_RB, HP), lambda i: (i, 0)),
        ],
        out_shape=[
            jax.ShapeDtypeStruct((N, HP), jnp.float32),
            jax.ShapeDtypeStruct((N, HP), jnp.float32),
        ],
    )(x, w1p, degp)


def _tc_d_body(a_ref, y_ref, dinv_ref, b1_ref, z_ref):
    dinv8 = dinv_ref[...]
    agg = a_ref[0] + a_ref[1] + y_ref[...]
    h = jnp.maximum(dinv8 * agg + b1_ref[...], 0.0)
    z_ref[...] = dinv8 * h


def _tc_d(acc1, y, dinv8, b1p):
    return pl.pallas_call(
        _tc_d_body,
        grid=(N // _RB,),
        in_specs=[
            pl.BlockSpec((NC, _RB, HP), lambda i: (0, i, 0)),
            pl.BlockSpec((_RB, HP), lambda i: (i, 0)),
            pl.BlockSpec((_RB, HP), lambda i: (i, 0)),
            pl.BlockSpec((1, HP), lambda i: (0, 0)),
        ],
        out_specs=pl.BlockSpec((_RB, HP), lambda i: (i, 0)),
        out_shape=jax.ShapeDtypeStruct((N, HP), jnp.float32),
    )(acc1, y, dinv8, b1p)


def _tc_f_body(a_ref, z_ref, dinv_ref, batch_ref, w2_ref, b2_ref, out_ref):
    q = dinv_ref[...] * (a_ref[0] + a_ref[1] + z_ref[...])
    gids = lax.broadcasted_iota(jnp.int32, (G, N), 0)
    onehot = (batch_ref[...] == gids).astype(jnp.float32)
    s = jnp.dot(onehot, q, preferred_element_type=jnp.float32)
    cnt = jnp.sum(onehot, axis=1, keepdims=True)
    pooled = s / jnp.maximum(cnt, 1.0)
    logits = jnp.dot(pooled[:, :6], w2_ref[...],
                     preferred_element_type=jnp.float32) + b2_ref[...]
    logits = jnp.where(cnt > 0.0, logits, 0.0)
    m = jnp.max(logits, axis=1, keepdims=True)
    e = logits - m
    out_ref[...] = e - jnp.log(jnp.sum(jnp.exp(e), axis=1, keepdims=True))


def _tc_f(acc2, z, dinv8, batch2d, w2, b2r):
    return pl.pallas_call(
        _tc_f_body,
        grid=(1,),
        in_specs=[
            pl.BlockSpec((NC, N, HP), lambda i: (0, 0, 0)),
            pl.BlockSpec((N, HP), lambda i: (0, 0)),
            pl.BlockSpec((N, HP), lambda i: (0, 0)),
            pl.BlockSpec((1, N), lambda i: (0, 0)),
            pl.BlockSpec((6, 10), lambda i: (0, 0)),
            pl.BlockSpec((1, 10), lambda i: (0, 0)),
        ],
        out_specs=pl.BlockSpec((G, 10), lambda i: (0, 0)),
        out_shape=jax.ShapeDtypeStruct((G, 10), jnp.float32),
    )(acc2, z, dinv8, batch2d, w2, b2r)


_PAD16 = 16


def kernel(x, edge_index, batch, W1, b1, W2, b2):
    e3 = edge_index.reshape(2, NW, EPW)
    e3 = jnp.pad(e3, ((0, 0), (0, 0), (0, EPWP - EPW)), constant_values=N)
    edges_r = e3.reshape(2, NW, NCH, K)

    zeros8 = jnp.zeros((NP, HP), jnp.float32)
    ones_k = jnp.ones((K, HP), jnp.float32)
    zpad = jnp.zeros((_PAD16, HP), jnp.float32)

    w1p = jnp.zeros((F, HP), jnp.float32).at[:, :6].set(W1)
    b1p = jnp.zeros((1, HP), jnp.float32).at[0, :6].set(b1)
    batch2d = batch.reshape(1, N)
    b2r = b2.reshape(1, 10)

    degp = _sc_deg(edges_r, ones_k, zeros8)
    y, dinv8 = _tc_b(x, w1p, degp)
    yp = jnp.concatenate([y, zpad], axis=0)
    acc1 = _sc_agg(edges_r, yp, zeros8)
    z = _tc_d(acc1, y, dinv8, b1p)
    zp = jnp.concatenate([z, zpad], axis=0)
    acc2 = _sc_agg(edges_r, zp, zeros8)
    return _tc_f(acc2, z, dinv8, batch2d, W2, b2r)


# speedup vs baseline: 79.8557x; 1.0980x over previous
import jax
import jax.numpy as jnp
from jax import lax
from jax.experimental import pallas as pl
from jax.experimental.pallas import tpu as pltpu
from jax.experimental.pallas import tpu_sc as plsc

N = 10000
E = 320000
F = 128
HP = 8
G = 8

NC = 2
NS = 16
NW = NC * NS
EPW = E // NW
K = 128
NCH = 79
EPWP = NCH * K
NP = N + 16
RPT = NP // NS
NBUF = 6
AHEAD = 3

_mesh = plsc.VectorSubcoreMesh(core_axis_name="c", subcore_axis_name="s")


def _make_sc_agg(with_gather: bool):
    def body(edges_hbm, table_hbm, zeros_hbm, out_hbm,
             row_all, col_all, msgs, acc, table_s, sem_g, sem_s):
        cid = lax.axis_index("c")
        sid = lax.axis_index("s")
        wid = cid * NS + sid

        pltpu.sync_copy(zeros_hbm.at[pl.ds(sid * RPT, RPT)],
                        acc.at[pl.ds(sid * RPT, RPT)])
        pltpu.sync_copy(edges_hbm.at[0, wid], row_all)
        pltpu.sync_copy(edges_hbm.at[1, wid], col_all)
        if with_gather:
            pltpu.sync_copy(table_hbm.at[pl.ds(sid * RPT, RPT)],
                            table_s.at[pl.ds(sid * RPT, RPT)])
        else:
            pltpu.sync_copy(table_hbm.at[pl.ds(0, K)], msgs.at[0])
        plsc.subcore_barrier()

        if with_gather:
            for p in range(AHEAD):
                pltpu.async_copy(table_s.at[row_all.at[p]], msgs.at[p],
                                 sem_g)

            def step(j, _):
                buf = lax.rem(j, NBUF)
                pltpu.make_async_copy(table_s.at[row_all.at[j]],
                                      msgs.at[buf], sem_g).wait()
                pltpu.async_copy(msgs.at[buf], acc.at[col_all.at[j]],
                                 sem_s, add=True)

                @pl.when(j >= AHEAD)
                def _():
                    old = lax.rem(j - AHEAD, NBUF)
                    pltpu.make_async_copy(msgs.at[old],
                                          acc.at[col_all.at[j - AHEAD]],
                                          sem_s).wait()

                @pl.when(j + AHEAD < NCH)
                def _():
                    nbuf = lax.rem(j + AHEAD, NBUF)
                    pltpu.async_copy(table_s.at[row_all.at[j + AHEAD]],
                                     msgs.at[nbuf], sem_g)
                return 0

            lax.fori_loop(0, NCH, step, 0)
            for t in range(NCH - AHEAD, NCH):
                pltpu.make_async_copy(msgs.at[t % NBUF],
                                      acc.at[col_all.at[t]], sem_s).wait()
        else:
            def step(j, _):
                pltpu.async_copy(msgs.at[0], acc.at[col_all.at[j]],
                                 sem_s, add=True)

                @pl.when(j >= AHEAD)
                def _():
                    pltpu.make_async_copy(msgs.at[0],
                                          acc.at[col_all.at[j - AHEAD]],
                                          sem_s).wait()
                return 0

            lax.fori_loop(0, NCH, step, 0)
            for t in range(NCH - AHEAD, NCH):
                pltpu.make_async_copy(msgs.at[0], acc.at[col_all.at[t]],
                                      sem_s).wait()

        plsc.subcore_barrier()
        pltpu.sync_copy(acc.at[pl.ds(sid * RPT, RPT)],
                        out_hbm.at[cid, pl.ds(sid * RPT, RPT)])

    return pl.kernel(
        body,
        out_type=jax.ShapeDtypeStruct((NC, NP, HP), jnp.float32),
        mesh=_mesh,
        scratch_types=[
            pltpu.VMEM((NCH, K), jnp.int32),
            pltpu.VMEM((NCH, K), jnp.int32),
            pltpu.VMEM((NBUF, K, HP), jnp.float32),
            pltpu.VMEM_SHARED((NP, HP), jnp.float32),
            pltpu.VMEM_SHARED((NP, HP), jnp.float32),
            pltpu.SemaphoreType.DMA,
            pltpu.SemaphoreType.DMA,
        ],
        compiler_params=pltpu.CompilerParams(use_tc_tiling_on_sc=False),
    )


_sc_agg = _make_sc_agg(True)
_sc_deg = _make_sc_agg(False)

HRPT = RPT // 2


def _sc_agg2z_body(edges_hbm, a1p_hbm, yp_hbm, dvp_hbm, b1v_hbm, zeros_hbm,
                   acc_out, z_out,
                   row_all, col_all, msgs, a0v, a1v, yv, dvv, zv, b1vv,
                   acc, table_s, sem_g, sem_s):
    cid = lax.axis_index("c")
    sid = lax.axis_index("s")
    wid = cid * NS + sid

    pltpu.sync_copy(zeros_hbm.at[pl.ds(sid * RPT, RPT)],
                    acc.at[pl.ds(sid * RPT, RPT)])
    pltpu.sync_copy(edges_hbm.at[0, wid], row_all)
    pltpu.sync_copy(edges_hbm.at[1, wid], col_all)
    pltpu.sync_copy(a1p_hbm.at[0, pl.ds(sid * RPT, RPT)], a0v)
    pltpu.sync_copy(a1p_hbm.at[1, pl.ds(sid * RPT, RPT)], a1v)
    pltpu.sync_copy(yp_hbm.at[pl.ds(sid * RPT, RPT)], yv)
    pltpu.sync_copy(dvp_hbm.at[pl.ds(sid * RPT, RPT)], dvv)
    pltpu.sync_copy(b1v_hbm, b1vv)

    b1vec = b1vv[...]
    lane = lax.iota(jnp.int32, 16)
    cc = jnp.bitwise_and(lane, 7)
    rr0 = lax.shift_right_logical(lane, 3)

    def ew(i, _):
        rr = rr0 + i + i
        a0 = plsc.load_gather(a0v, [rr, cc])
        a1 = plsc.load_gather(a1v, [rr, cc])
        yy = plsc.load_gather(yv, [rr, cc])
        dv = plsc.load_gather(dvv, [rr, cc])
        h = jnp.maximum(dv * (a0 + a1 + yy) + b1vec, 0.0)
        plsc.store_scatter(zv, [rr, cc], dv * h)
        return 0

    lax.fori_loop(0, RPT // 2, ew, 0)

    pltpu.sync_copy(zv, table_s.at[pl.ds(sid * RPT, RPT)])
    pltpu.sync_copy(zv.at[pl.ds(cid * HRPT, HRPT)],
                    z_out.at[pl.ds(sid * RPT + cid * HRPT, HRPT)])
    plsc.subcore_barrier()

    for p in range(AHEAD):
        pltpu.async_copy(table_s.at[row_all.at[p]], msgs.at[p], sem_g)

    def step(j, _):
        buf = lax.rem(j, NBUF)
        pltpu.make_async_copy(table_s.at[row_all.at[j]],
                              msgs.at[buf], sem_g).wait()
        pltpu.async_copy(msgs.at[buf], acc.at[col_all.at[j]],
                         sem_s, add=True)

        @pl.when(j >= AHEAD)
        def _():
            old = lax.rem(j - AHEAD, NBUF)
            pltpu.make_async_copy(msgs.at[old],
                                  acc.at[col_all.at[j - AHEAD]],
                                  sem_s).wait()

        @pl.when(j + AHEAD < NCH)
        def _():
            nbuf = lax.rem(j + AHEAD, NBUF)
            pltpu.async_copy(table_s.at[row_all.at[j + AHEAD]],
                             msgs.at[nbuf], sem_g)
        return 0

    lax.fori_loop(0, NCH, step, 0)
    for t in range(NCH - AHEAD, NCH):
        pltpu.make_async_copy(msgs.at[t % NBUF],
                              acc.at[col_all.at[t]], sem_s).wait()

    plsc.subcore_barrier()
    pltpu.sync_copy(acc.at[pl.ds(sid * RPT, RPT)],
                    acc_out.at[cid, pl.ds(sid * RPT, RPT)])


_sc_agg2z = pl.kernel(
    _sc_agg2z_body,
    out_type=[
        jax.ShapeDtypeStruct((NC, NP, HP), jnp.float32),
        jax.ShapeDtypeStruct((NP, HP), jnp.float32),
    ],
    mesh=_mesh,
    scratch_types=[
        pltpu.VMEM((NCH, K), jnp.int32),
        pltpu.VMEM((NCH, K), jnp.int32),
        pltpu.VMEM((NBUF, K, HP), jnp.float32),
        pltpu.VMEM((RPT, HP), jnp.float32),
        pltpu.VMEM((RPT, HP), jnp.float32),
        pltpu.VMEM((RPT, HP), jnp.float32),
        pltpu.VMEM((RPT, HP), jnp.float32),
        pltpu.VMEM((RPT, HP), jnp.float32),
        pltpu.VMEM((16,), jnp.float32),
        pltpu.VMEM_SHARED((NP, HP), jnp.float32),
        pltpu.VMEM_SHARED((NP, HP), jnp.float32),
        pltpu.SemaphoreType.DMA,
        pltpu.SemaphoreType.DMA,
    ],
    compiler_params=pltpu.CompilerParams(use_tc_tiling_on_sc=False,
                                         needs_layout_passes=False),
)


_RB = 1000


def _tc_b_body(x_ref, w_ref, degp_ref, y_ref, dinv_ref):
    xw = jnp.dot(x_ref[...], w_ref[...], preferred_element_type=jnp.float32)
    deg = degp_ref[0, :, 0] + degp_ref[1, :, 0] + 1.0
    dinv = lax.rsqrt(deg)[:, None]
    dinv8 = jnp.broadcast_to(dinv, (x_ref.shape[0], HP))
    dinv_ref[...] = dinv8
    y_ref[...] = xw * dinv8


def _tc_b(x, w1p, degp):
    return pl.pallas_call(
        _tc_b_body,
        grid=(N // _RB,),
        in_specs=[
            pl.BlockSpec((_RB, F), lambda i: (i, 0)),
            pl.BlockSpec((F, HP), lambda i: (0, 0)),
            pl.BlockSpec((NC, _RB, HP), lambda i: (0, i, 0)),
        ],
        out_specs=[
            pl.BlockSpec((_RB, HP), lambda i: (i, 0)),
            pl.BlockSpec((_RB, HP), lambda i: (i, 0)),
        ],
        out_shape=[
            jax.ShapeDtypeStruct((N, HP), jnp.float32),
            jax.ShapeDtypeStruct((N, HP), jnp.float32),
        ],
    )(x, w1p, degp)


def _tc_d_body(a_ref, y_ref, dinv_ref, b1_ref, z_ref):
    dinv8 = dinv_ref[...]
    agg = a_ref[0] + a_ref[1] + y_ref[...]
    h = jnp.maximum(dinv8 * agg + b1_ref[...], 0.0)
    z_ref[...] = dinv8 * h


def _tc_d(acc1, y, dinv8, b1p):
    return pl.pallas_call(
        _tc_d_body,
        grid=(N // _RB,),
        in_specs=[
            pl.BlockSpec((NC, _RB, HP), lambda i: (0, i, 0)),
            pl.BlockSpec((_RB, HP), lambda i: (i, 0)),
            pl.BlockSpec((_RB, HP), lambda i: (i, 0)),
            pl.BlockSpec((1, HP), lambda i: (0, 0)),
        ],
        out_specs=pl.BlockSpec((_RB, HP), lambda i: (i, 0)),
        out_shape=jax.ShapeDtypeStruct((N, HP), jnp.float32),
    )(acc1, y, dinv8, b1p)


def _tc_f_body(a_ref, z_ref, dinv_ref, batch_ref, w2_ref, b2_ref, out_ref):
    q = dinv_ref[...] * (a_ref[0] + a_ref[1] + z_ref[...])
    gids = lax.broadcasted_iota(jnp.int32, (G, N), 0)
    onehot = (batch_ref[...] == gids).astype(jnp.float32)
    s = jnp.dot(onehot, q, preferred_element_type=jnp.float32)
    cnt = jnp.sum(onehot, axis=1, keepdims=True)
    pooled = s / jnp.maximum(cnt, 1.0)
    logits = jnp.dot(pooled[:, :6], w2_ref[...],
                     preferred_element_type=jnp.float32) + b2_ref[...]
    logits = jnp.where(cnt > 0.0, logits, 0.0)
    m = jnp.max(logits, axis=1, keepdims=True)
    e = logits - m
    out_ref[...] = e - jnp.log(jnp.sum(jnp.exp(e), axis=1, keepdims=True))


def _tc_f(acc2, z, dinv8, batch2d, w2, b2r):
    return pl.pallas_call(
        _tc_f_body,
        grid=(1,),
        in_specs=[
            pl.BlockSpec((NC, N, HP), lambda i: (0, 0, 0)),
            pl.BlockSpec((N, HP), lambda i: (0, 0)),
            pl.BlockSpec((N, HP), lambda i: (0, 0)),
            pl.BlockSpec((1, N), lambda i: (0, 0)),
            pl.BlockSpec((6, 10), lambda i: (0, 0)),
            pl.BlockSpec((1, 10), lambda i: (0, 0)),
        ],
        out_specs=pl.BlockSpec((G, 10), lambda i: (0, 0)),
        out_shape=jax.ShapeDtypeStruct((G, 10), jnp.float32),
    )(acc2, z, dinv8, batch2d, w2, b2r)


_PAD16 = 16


def kernel(x, edge_index, batch, W1, b1, W2, b2):
    e3 = edge_index.reshape(2, NW, EPW)
    e3 = jnp.pad(e3, ((0, 0), (0, 0), (0, EPWP - EPW)), constant_values=N)
    edges_r = e3.reshape(2, NW, NCH, K)

    zeros8 = jnp.zeros((NP, HP), jnp.float32)
    ones_k = jnp.ones((K, HP), jnp.float32)
    zpad = jnp.zeros((_PAD16, HP), jnp.float32)

    w1p = jnp.zeros((F, HP), jnp.float32).at[:, :6].set(W1)
    b1p = jnp.zeros((1, HP), jnp.float32).at[0, :6].set(b1)
    batch2d = batch.reshape(1, N)
    b2r = b2.reshape(1, 10)

    b1v16 = jnp.concatenate([b1p[0], b1p[0]])

    degp = _sc_deg(edges_r, ones_k, zeros8)
    y, dinv8 = _tc_b(x, w1p, degp)
    yp = jnp.concatenate([y, zpad], axis=0)
    dvp = jnp.concatenate([dinv8, zpad], axis=0)
    acc1 = _sc_agg(edges_r, yp, zeros8)
    acc2, zp = _sc_agg2z(edges_r, acc1, yp, dvp, b1v16, zeros8)
    return _tc_f(acc2, zp, dinv8, batch2d, W2, b2r)


# speedup vs baseline: 85.6811x; 1.0729x over previous
import jax
import jax.numpy as jnp
from jax import lax
from jax.experimental import pallas as pl
from jax.experimental.pallas import tpu as pltpu
from jax.experimental.pallas import tpu_sc as plsc

N = 10000
E = 320000
F = 128
HP = 8
G = 8

NC = 2
NS = 16
NW = NC * NS
EPW = E // NW
K = 128
NCH = 79
EPWP = NCH * K
NP = N + 16
RPT = NP // NS
NBUF = 6
AHEAD = 3

_mesh = plsc.VectorSubcoreMesh(core_axis_name="c", subcore_axis_name="s")


def _make_sc_agg(with_gather: bool):
    def body(edges_hbm, table_hbm, zeros_hbm, out_hbm,
             row_all, col_all, msgs, acc, table_s, sem_g, sem_s):
        cid = lax.axis_index("c")
        sid = lax.axis_index("s")
        wid = cid * NS + sid

        pltpu.sync_copy(zeros_hbm.at[pl.ds(sid * RPT, RPT)],
                        acc.at[pl.ds(sid * RPT, RPT)])
        pltpu.sync_copy(edges_hbm.at[0, wid], row_all)
        pltpu.sync_copy(edges_hbm.at[1, wid], col_all)
        if with_gather:
            pltpu.sync_copy(table_hbm.at[pl.ds(sid * RPT, RPT)],
                            table_s.at[pl.ds(sid * RPT, RPT)])
        else:
            pltpu.sync_copy(table_hbm.at[pl.ds(0, K)], msgs.at[0])
        plsc.subcore_barrier()

        if with_gather:
            for p in range(AHEAD):
                pltpu.async_copy(table_s.at[row_all.at[p]], msgs.at[p],
                                 sem_g)

            def step(j, _):
                buf = lax.rem(j, NBUF)
                pltpu.make_async_copy(table_s.at[row_all.at[j]],
                                      msgs.at[buf], sem_g).wait()
                pltpu.async_copy(msgs.at[buf], acc.at[col_all.at[j]],
                                 sem_s, add=True)

                @pl.when(j >= AHEAD)
                def _():
                    old = lax.rem(j - AHEAD, NBUF)
                    pltpu.make_async_copy(msgs.at[old],
                                          acc.at[col_all.at[j - AHEAD]],
                                          sem_s).wait()

                @pl.when(j + AHEAD < NCH)
                def _():
                    nbuf = lax.rem(j + AHEAD, NBUF)
                    pltpu.async_copy(table_s.at[row_all.at[j + AHEAD]],
                                     msgs.at[nbuf], sem_g)
                return 0

            lax.fori_loop(0, NCH, step, 0)
            for t in range(NCH - AHEAD, NCH):
                pltpu.make_async_copy(msgs.at[t % NBUF],
                                      acc.at[col_all.at[t]], sem_s).wait()
        else:
            def step(j, _):
                pltpu.async_copy(msgs.at[0], acc.at[col_all.at[j]],
                                 sem_s, add=True)

                @pl.when(j >= AHEAD)
                def _():
                    pltpu.make_async_copy(msgs.at[0],
                                          acc.at[col_all.at[j - AHEAD]],
                                          sem_s).wait()
                return 0

            lax.fori_loop(0, NCH, step, 0)
            for t in range(NCH - AHEAD, NCH):
                pltpu.make_async_copy(msgs.at[0], acc.at[col_all.at[t]],
                                      sem_s).wait()

        plsc.subcore_barrier()
        pltpu.sync_copy(acc.at[pl.ds(sid * RPT, RPT)],
                        out_hbm.at[cid, pl.ds(sid * RPT, RPT)])

    return pl.kernel(
        body,
        out_type=jax.ShapeDtypeStruct((NC, NP, HP), jnp.float32),
        mesh=_mesh,
        scratch_types=[
            pltpu.VMEM((NCH, K), jnp.int32),
            pltpu.VMEM((NCH, K), jnp.int32),
            pltpu.VMEM((NBUF, K, HP), jnp.float32),
            pltpu.VMEM_SHARED((NP, HP), jnp.float32),
            pltpu.VMEM_SHARED((NP, HP), jnp.float32),
            pltpu.SemaphoreType.DMA,
            pltpu.SemaphoreType.DMA,
        ],
        compiler_params=pltpu.CompilerParams(use_tc_tiling_on_sc=False),
    )


_sc_agg = _make_sc_agg(True)
_sc_deg = _make_sc_agg(False)

HRPT = RPT // 2


def _sc_agg2z_body(edges_hbm, a1p_hbm, yp_hbm, dvp_hbm, b1v_hbm, zeros_hbm,
                   acc_out, z_out,
                   row_all, col_all, msgs, a0v, a1v, yv, dvv, zv, b1vv,
                   acc, table_s, sem_g, sem_s):
    cid = lax.axis_index("c")
    sid = lax.axis_index("s")
    wid = cid * NS + sid

    pltpu.sync_copy(zeros_hbm.at[pl.ds(sid * RPT, RPT)],
                    acc.at[pl.ds(sid * RPT, RPT)])
    pltpu.sync_copy(edges_hbm.at[0, wid], row_all)
    pltpu.sync_copy(edges_hbm.at[1, wid], col_all)
    pltpu.sync_copy(a1p_hbm.at[0, pl.ds(sid * RPT, RPT)], a0v)
    pltpu.sync_copy(a1p_hbm.at[1, pl.ds(sid * RPT, RPT)], a1v)
    pltpu.sync_copy(yp_hbm.at[pl.ds(sid * RPT, RPT)], yv)
    pltpu.sync_copy(dvp_hbm.at[pl.ds(sid * RPT, RPT)], dvv)
    pltpu.sync_copy(b1v_hbm, b1vv)

    b1vec = b1vv[...]
    lane = lax.iota(jnp.int32, 16)
    cc = jnp.bitwise_and(lane, 7)
    rr0 = lax.shift_right_logical(lane, 3)

    def ew(i, _):
        rr = rr0 + i + i
        a0 = plsc.load_gather(a0v, [rr, cc])
        a1 = plsc.load_gather(a1v, [rr, cc])
        yy = plsc.load_gather(yv, [rr, cc])
        dv = plsc.load_gather(dvv, [rr, cc])
        h = jnp.maximum(dv * (a0 + a1 + yy) + b1vec, 0.0)
        plsc.store_scatter(zv, [rr, cc], dv * h)
        return 0

    lax.fori_loop(0, RPT // 2, ew, 0)

    pltpu.sync_copy(zv, table_s.at[pl.ds(sid * RPT, RPT)])
    pltpu.sync_copy(zv.at[pl.ds(cid * HRPT, HRPT)],
                    z_out.at[pl.ds(sid * RPT + cid * HRPT, HRPT)])
    plsc.subcore_barrier()

    for p in range(AHEAD):
        pltpu.async_copy(table_s.at[row_all.at[p]], msgs.at[p], sem_g)

    def step(j, _):
        buf = lax.rem(j, NBUF)
        pltpu.make_async_copy(table_s.at[row_all.at[j]],
                              msgs.at[buf], sem_g).wait()
        pltpu.async_copy(msgs.at[buf], acc.at[col_all.at[j]],
                         sem_s, add=True)

        @pl.when(j >= AHEAD)
        def _():
            old = lax.rem(j - AHEAD, NBUF)
            pltpu.make_async_copy(msgs.at[old],
                                  acc.at[col_all.at[j - AHEAD]],
                                  sem_s).wait()

        @pl.when(j + AHEAD < NCH)
        def _():
            nbuf = lax.rem(j + AHEAD, NBUF)
            pltpu.async_copy(table_s.at[row_all.at[j + AHEAD]],
                             msgs.at[nbuf], sem_g)
        return 0

    lax.fori_loop(0, NCH, step, 0)
    for t in range(NCH - AHEAD, NCH):
        pltpu.make_async_copy(msgs.at[t % NBUF],
                              acc.at[col_all.at[t]], sem_s).wait()

    plsc.subcore_barrier()
    pltpu.sync_copy(acc.at[pl.ds(sid * RPT, RPT)],
                    acc_out.at[cid, pl.ds(sid * RPT, RPT)])


_sc_agg2z = pl.kernel(
    _sc_agg2z_body,
    out_type=[
        jax.ShapeDtypeStruct((NC, NP, HP), jnp.float32),
        jax.ShapeDtypeStruct((NP, HP), jnp.float32),
    ],
    mesh=_mesh,
    scratch_types=[
        pltpu.VMEM((NCH, K), jnp.int32),
        pltpu.VMEM((NCH, K), jnp.int32),
        pltpu.VMEM((NBUF, K, HP), jnp.float32),
        pltpu.VMEM((RPT, HP), jnp.float32),
        pltpu.VMEM((RPT, HP), jnp.float32),
        pltpu.VMEM((RPT, HP), jnp.float32),
        pltpu.VMEM((RPT, HP), jnp.float32),
        pltpu.VMEM((RPT, HP), jnp.float32),
        pltpu.VMEM((16,), jnp.float32),
        pltpu.VMEM_SHARED((NP, HP), jnp.float32),
        pltpu.VMEM_SHARED((NP, HP), jnp.float32),
        pltpu.SemaphoreType.DMA,
        pltpu.SemaphoreType.DMA,
    ],
    compiler_params=pltpu.CompilerParams(use_tc_tiling_on_sc=False,
                                         needs_layout_passes=False),
)


def _sc_agg1y_body(edges_hbm, degp_hbm, xwp_hbm, zeros_hbm,
                   acc_out, y_out, dv_out,
                   row_all, col_all, msgs, a0v, a1v, yv, zv, dvv2,
                   acc, table_s, sem_g, sem_s):
    cid = lax.axis_index("c")
    sid = lax.axis_index("s")
    wid = cid * NS + sid

    pltpu.sync_copy(zeros_hbm.at[pl.ds(sid * RPT, RPT)],
                    acc.at[pl.ds(sid * RPT, RPT)])
    pltpu.sync_copy(edges_hbm.at[0, wid], row_all)
    pltpu.sync_copy(edges_hbm.at[1, wid], col_all)
    pltpu.sync_copy(degp_hbm.at[0, pl.ds(sid * RPT, RPT)], a0v)
    pltpu.sync_copy(degp_hbm.at[1, pl.ds(sid * RPT, RPT)], a1v)
    pltpu.sync_copy(xwp_hbm.at[pl.ds(sid * RPT, RPT)], yv)

    lane = lax.iota(jnp.int32, 16)
    cc = jnp.bitwise_and(lane, 7)
    rr0 = lax.shift_right_logical(lane, 3)

    def ew(i, _):
        rr = rr0 + i + i
        d0 = plsc.load_gather(a0v, [rr, cc])
        d1 = plsc.load_gather(a1v, [rr, cc])
        xw = plsc.load_gather(yv, [rr, cc])
        x = d0 + d1 + 1.0
        ib = plsc.bitcast(x, jnp.int32)
        seed = 0x5F3759DF - lax.shift_right_logical(ib, 1)
        r = plsc.bitcast(seed, jnp.float32)
        half = 0.5 * x
        r = r * (1.5 - half * r * r)
        r = r * (1.5 - half * r * r)
        r = r * (1.5 - half * r * r)
        r = r * (1.5 - half * r * r)
        plsc.store_scatter(dvv2, [rr, cc], r)
        plsc.store_scatter(zv, [rr, cc], r * xw)
        return 0

    lax.fori_loop(0, RPT // 2, ew, 0)

    pltpu.sync_copy(zv, table_s.at[pl.ds(sid * RPT, RPT)])
    pltpu.sync_copy(zv.at[pl.ds(cid * HRPT, HRPT)],
                    y_out.at[pl.ds(sid * RPT + cid * HRPT, HRPT)])
    pltpu.sync_copy(dvv2.at[pl.ds(cid * HRPT, HRPT)],
                    dv_out.at[pl.ds(sid * RPT + cid * HRPT, HRPT)])
    plsc.subcore_barrier()

    for p in range(AHEAD):
        pltpu.async_copy(table_s.at[row_all.at[p]], msgs.at[p], sem_g)

    def step(j, _):
        buf = lax.rem(j, NBUF)
        pltpu.make_async_copy(table_s.at[row_all.at[j]],
                              msgs.at[buf], sem_g).wait()
        pltpu.async_copy(msgs.at[buf], acc.at[col_all.at[j]],
                         sem_s, add=True)

        @pl.when(j >= AHEAD)
        def _():
            old = lax.rem(j - AHEAD, NBUF)
            pltpu.make_async_copy(msgs.at[old],
                                  acc.at[col_all.at[j - AHEAD]],
                                  sem_s).wait()

        @pl.when(j + AHEAD < NCH)
        def _():
            nbuf = lax.rem(j + AHEAD, NBUF)
            pltpu.async_copy(table_s.at[row_all.at[j + AHEAD]],
                             msgs.at[nbuf], sem_g)
        return 0

    lax.fori_loop(0, NCH, step, 0)
    for t in range(NCH - AHEAD, NCH):
        pltpu.make_async_copy(msgs.at[t % NBUF],
                              acc.at[col_all.at[t]], sem_s).wait()

    plsc.subcore_barrier()
    pltpu.sync_copy(acc.at[pl.ds(sid * RPT, RPT)],
                    acc_out.at[cid, pl.ds(sid * RPT, RPT)])


_sc_agg1y = pl.kernel(
    _sc_agg1y_body,
    out_type=[
        jax.ShapeDtypeStruct((NC, NP, HP), jnp.float32),
        jax.ShapeDtypeStruct((NP, HP), jnp.float32),
        jax.ShapeDtypeStruct((NP, HP), jnp.float32),
    ],
    mesh=_mesh,
    scratch_types=[
        pltpu.VMEM((NCH, K), jnp.int32),
        pltpu.VMEM((NCH, K), jnp.int32),
        pltpu.VMEM((NBUF, K, HP), jnp.float32),
        pltpu.VMEM((RPT, HP), jnp.float32),
        pltpu.VMEM((RPT, HP), jnp.float32),
        pltpu.VMEM((RPT, HP), jnp.float32),
        pltpu.VMEM((RPT, HP), jnp.float32),
        pltpu.VMEM((RPT, HP), jnp.float32),
        pltpu.VMEM_SHARED((NP, HP), jnp.float32),
        pltpu.VMEM_SHARED((NP, HP), jnp.float32),
        pltpu.SemaphoreType.DMA,
        pltpu.SemaphoreType.DMA,
    ],
    compiler_params=pltpu.CompilerParams(use_tc_tiling_on_sc=False,
                                         needs_layout_passes=False),
)


_RB = 1000


def _tc_b_body(x_ref, w_ref, y_ref):
    y_ref[...] = jnp.dot(x_ref[...], w_ref[...],
                         preferred_element_type=jnp.float32)


def _tc_b(x, w1p):
    return pl.pallas_call(
        _tc_b_body,
        grid=(N // _RB,),
        in_specs=[
            pl.BlockSpec((_RB, F), lambda i: (i, 0)),
            pl.BlockSpec((F, HP), lambda i: (0, 0)),
        ],
        out_specs=pl.BlockSpec((_RB, HP), lambda i: (i, 0)),
        out_shape=jax.ShapeDtypeStruct((N, HP), jnp.float32),
    )(x, w1p)


def _tc_d_body(a_ref, y_ref, dinv_ref, b1_ref, z_ref):
    dinv8 = dinv_ref[...]
    agg = a_ref[0] + a_ref[1] + y_ref[...]
    h = jnp.maximum(dinv8 * agg + b1_ref[...], 0.0)
    z_ref[...] = dinv8 * h


def _tc_d(acc1, y, dinv8, b1p):
    return pl.pallas_call(
        _tc_d_body,
        grid=(N // _RB,),
        in_specs=[
            pl.BlockSpec((NC, _RB, HP), lambda i: (0, i, 0)),
            pl.BlockSpec((_RB, HP), lambda i: (i, 0)),
            pl.BlockSpec((_RB, HP), lambda i: (i, 0)),
            pl.BlockSpec((1, HP), lambda i: (0, 0)),
        ],
        out_specs=pl.BlockSpec((_RB, HP), lambda i: (i, 0)),
        out_shape=jax.ShapeDtypeStruct((N, HP), jnp.float32),
    )(acc1, y, dinv8, b1p)


def _tc_f_body(a_ref, z_ref, dinv_ref, batch_ref, w2_ref, b2_ref, out_ref):
    q = dinv_ref[...] * (a_ref[0] + a_ref[1] + z_ref[...])
    gids = lax.broadcasted_iota(jnp.int32, (G, N), 0)
    onehot = (batch_ref[...] == gids).astype(jnp.float32)
    s = jnp.dot(onehot, q, preferred_element_type=jnp.float32)
    cnt = jnp.sum(onehot, axis=1, keepdims=True)
    pooled = s / jnp.maximum(cnt, 1.0)
    logits = jnp.dot(pooled[:, :6], w2_ref[...],
                     preferred_element_type=jnp.float32) + b2_ref[...]
    logits = jnp.where(cnt > 0.0, logits, 0.0)
    m = jnp.max(logits, axis=1, keepdims=True)
    e = logits - m
    out_ref[...] = e - jnp.log(jnp.sum(jnp.exp(e), axis=1, keepdims=True))


def _tc_f(acc2, z, dinv8, batch2d, w2, b2r):
    return pl.pallas_call(
        _tc_f_body,
        grid=(1,),
        in_specs=[
            pl.BlockSpec((NC, N, HP), lambda i: (0, 0, 0)),
            pl.BlockSpec((N, HP), lambda i: (0, 0)),
            pl.BlockSpec((N, HP), lambda i: (0, 0)),
            pl.BlockSpec((1, N), lambda i: (0, 0)),
            pl.BlockSpec((6, 10), lambda i: (0, 0)),
            pl.BlockSpec((1, 10), lambda i: (0, 0)),
        ],
        out_specs=pl.BlockSpec((G, 10), lambda i: (0, 0)),
        out_shape=jax.ShapeDtypeStruct((G, 10), jnp.float32),
    )(acc2, z, dinv8, batch2d, w2, b2r)


_PAD16 = 16


def kernel(x, edge_index, batch, W1, b1, W2, b2):
    e3 = edge_index.reshape(2, NW, EPW)
    e3 = jnp.pad(e3, ((0, 0), (0, 0), (0, EPWP - EPW)), constant_values=N)
    edges_r = e3.reshape(2, NW, NCH, K)

    zeros8 = jnp.zeros((NP, HP), jnp.float32)
    ones_k = jnp.ones((K, HP), jnp.float32)
    zpad = jnp.zeros((_PAD16, HP), jnp.float32)

    w1p = jnp.zeros((F, HP), jnp.float32).at[:, :6].set(W1)
    b1p = jnp.zeros((1, HP), jnp.float32).at[0, :6].set(b1)
    batch2d = batch.reshape(1, N)
    b2r = b2.reshape(1, 10)

    b1v16 = jnp.concatenate([b1p[0], b1p[0]])

    xw = _tc_b(x, w1p)
    degp = _sc_deg(edges_r, ones_k, zeros8)
    xwp = jnp.concatenate([xw, zpad], axis=0)
    acc1, yp, dvp = _sc_agg1y(edges_r, degp, xwp, zeros8)
    acc2, zp = _sc_agg2z(edges_r, acc1, yp, dvp, b1v16, zeros8)
    return _tc_f(acc2, zp, dvp, batch2d, W2, b2r)
